# initial kernel scaffold (unmeasured)
import jax
import jax.numpy as jnp
from jax import lax
from jax.experimental import pallas as pl
from jax.experimental.pallas import tpu as pltpu

Y = 4


def kernel(x, dy, gamma):
    del gamma
    m, d = x.shape

    def body(x_ref, dy_ref, out_ref, self_ref, comm_ref, send_sems, recv_sems):
        my_x = lax.axis_index("x")
        my_y = lax.axis_index("y")
        my_z = lax.axis_index("z")

        xv = x_ref[:, :]
        dyv = dy_ref[:, :]
        mu = jnp.mean(xv, axis=1, keepdims=True)
        diff = xv - mu
        var = jnp.mean(diff * diff, axis=1, keepdims=True)
        rstd = lax.rsqrt(var + 1e-5)
        self_ref[0, :] = jnp.sum(dyv * (diff * rstd), axis=0)
        self_ref[1, :] = jnp.sum(dyv, axis=0)

        sends = []
        for k in range(1, Y):
            tgt_y = lax.rem(my_y + k, Y)
            rdma = pltpu.make_async_remote_copy(
                src_ref=self_ref,
                dst_ref=comm_ref.at[Y - 1 - k],
                send_sem=send_sems.at[k - 1],
                recv_sem=recv_sems.at[Y - 1 - k],
                device_id=(my_x, tgt_y, my_z),
                device_id_type=pl.DeviceIdType.MESH,
            )
            rdma.start()
            sends.append(rdma)

        for r in sends:
            r.wait_recv()
        acc = self_ref[:, :]
        for s in range(Y - 1):
            acc = acc + comm_ref[s, :, :]
        out_ref[:, :] = acc
        for r in sends:
            r.wait_send()

    return pl.pallas_call(
        body,
        out_shape=jax.ShapeDtypeStruct((2, d), jnp.float32),
        in_specs=[
            pl.BlockSpec(memory_space=pltpu.VMEM),
            pl.BlockSpec(memory_space=pltpu.VMEM),
        ],
        out_specs=pl.BlockSpec(memory_space=pltpu.VMEM),
        scratch_shapes=[
            pltpu.VMEM((2, d), jnp.float32),
            pltpu.VMEM((Y - 1, 2, d), jnp.float32),
            pltpu.SemaphoreType.DMA((Y - 1,)),
            pltpu.SemaphoreType.DMA((Y - 1,)),
        ],
    )(x, dy)


# baseline (device time: 22410 ns/iter reference)
import jax
import jax.numpy as jnp
from jax import lax
from jax.experimental import pallas as pl
from jax.experimental.pallas import tpu as pltpu

Y = 4
BM = 256


def kernel(x, dy, gamma):
    del gamma
    m, d = x.shape

    def body(x_ref, dy_ref, out_ref, self_ref, comm_ref, send_sems, recv_sems):
        step = pl.program_id(0)
        nsteps = pl.num_programs(0)

        xv = x_ref[:, :]
        dyv = dy_ref[:, :]
        mu = jnp.mean(xv, axis=1, keepdims=True)
        diff = xv - mu
        var = jnp.mean(diff * diff, axis=1, keepdims=True)
        rstd = lax.rsqrt(var + 1e-5)
        dgamma = jnp.sum(dyv * (diff * rstd), axis=0)
        dbeta = jnp.sum(dyv, axis=0)

        @pl.when(step == 0)
        def _():
            self_ref[0, :] = dgamma
            self_ref[1, :] = dbeta

        @pl.when(step != 0)
        def _():
            self_ref[0, :] = self_ref[0, :] + dgamma
            self_ref[1, :] = self_ref[1, :] + dbeta

        @pl.when(step == nsteps - 1)
        def _():
            my_x = lax.axis_index("x")
            my_y = lax.axis_index("y")
            my_z = lax.axis_index("z")

            barrier_sem = pltpu.get_barrier_semaphore()
            for k in range(1, Y):
                pl.semaphore_signal(
                    barrier_sem,
                    inc=1,
                    device_id=(my_x, lax.rem(my_y + k, Y), my_z),
                    device_id_type=pl.DeviceIdType.MESH,
                )
            pl.semaphore_wait(barrier_sem, Y - 1)

            sends = []
            for k in range(1, Y):
                tgt_y = lax.rem(my_y + k, Y)
                rdma = pltpu.make_async_remote_copy(
                    src_ref=self_ref,
                    dst_ref=comm_ref.at[Y - 1 - k],
                    send_sem=send_sems.at[k - 1],
                    recv_sem=recv_sems.at[Y - 1 - k],
                    device_id=(my_x, tgt_y, my_z),
                    device_id_type=pl.DeviceIdType.MESH,
                )
                rdma.start()
                sends.append(rdma)

            for r in sends:
                r.wait_recv()
            acc = self_ref[:, :]
            for s in range(Y - 1):
                acc = acc + comm_ref[s, :, :]
            out_ref[:, :] = acc
            for r in sends:
                r.wait_send()

    return pl.pallas_call(
        body,
        grid=(m // BM,),
        out_shape=jax.ShapeDtypeStruct((2, d), jnp.float32),
        in_specs=[
            pl.BlockSpec((BM, d), lambda i: (i, 0)),
            pl.BlockSpec((BM, d), lambda i: (i, 0)),
        ],
        out_specs=pl.BlockSpec((2, d), lambda i: (0, 0)),
        scratch_shapes=[
            pltpu.VMEM((2, d), jnp.float32),
            pltpu.VMEM((Y - 1, 2, d), jnp.float32),
            pltpu.SemaphoreType.DMA((Y - 1,)),
            pltpu.SemaphoreType.DMA((Y - 1,)),
        ],
        compiler_params=pltpu.CompilerParams(collective_id=0),
    )(x, dy)


# device time: 20848 ns/iter; 1.0749x vs baseline; 1.0749x over previous
import jax
import jax.numpy as jnp
from jax import lax
from jax.experimental import pallas as pl
from jax.experimental.pallas import tpu as pltpu

Y = 4
BM = 512


def kernel(x, dy, gamma):
    del gamma
    m, d = x.shape

    def body(x_ref, dy_ref, out_ref, self_ref, comm_ref, send_sems, recv_sems):
        step = pl.program_id(0)
        nsteps = pl.num_programs(0)

        xv = x_ref[:, :]
        dyv = dy_ref[:, :]
        dd = xv.shape[1]
        s1 = jnp.sum(xv, axis=1, keepdims=True)
        s2 = jnp.sum(xv * xv, axis=1, keepdims=True)
        mu = s1 * (1.0 / dd)
        var = s2 * (1.0 / dd) - mu * mu
        rstd = lax.rsqrt(var + 1e-5)
        xhat = xv * rstd - mu * rstd
        dgamma = jnp.sum(dyv * xhat, axis=0)
        dbeta = jnp.sum(dyv, axis=0)

        @pl.when(step == 0)
        def _():
            self_ref[0, :] = dgamma
            self_ref[1, :] = dbeta

        @pl.when(step != 0)
        def _():
            self_ref[0, :] = self_ref[0, :] + dgamma
            self_ref[1, :] = self_ref[1, :] + dbeta

        @pl.when(step == nsteps - 1)
        def _():
            my_x = lax.axis_index("x")
            my_y = lax.axis_index("y")
            my_z = lax.axis_index("z")

            barrier_sem = pltpu.get_barrier_semaphore()
            for k in range(1, Y):
                pl.semaphore_signal(
                    barrier_sem,
                    inc=1,
                    device_id=(my_x, lax.rem(my_y + k, Y), my_z),
                    device_id_type=pl.DeviceIdType.MESH,
                )
            pl.semaphore_wait(barrier_sem, Y - 1)

            sends = []
            for k in range(1, Y):
                tgt_y = lax.rem(my_y + k, Y)
                rdma = pltpu.make_async_remote_copy(
                    src_ref=self_ref,
                    dst_ref=comm_ref.at[Y - 1 - k],
                    send_sem=send_sems.at[k - 1],
                    recv_sem=recv_sems.at[Y - 1 - k],
                    device_id=(my_x, tgt_y, my_z),
                    device_id_type=pl.DeviceIdType.MESH,
                )
                rdma.start()
                sends.append(rdma)

            for r in sends:
                r.wait_recv()
            acc = self_ref[:, :]
            for s in range(Y - 1):
                acc = acc + comm_ref[s, :, :]
            out_ref[:, :] = acc
            for r in sends:
                r.wait_send()

    return pl.pallas_call(
        body,
        grid=(m // BM,),
        out_shape=jax.ShapeDtypeStruct((2, d), jnp.float32),
        in_specs=[
            pl.BlockSpec((BM, d), lambda i: (i, 0)),
            pl.BlockSpec((BM, d), lambda i: (i, 0)),
        ],
        out_specs=pl.BlockSpec((2, d), lambda i: (0, 0)),
        scratch_shapes=[
            pltpu.VMEM((2, d), jnp.float32),
            pltpu.VMEM((Y - 1, 2, d), jnp.float32),
            pltpu.SemaphoreType.DMA((Y - 1,)),
            pltpu.SemaphoreType.DMA((Y - 1,)),
        ],
        compiler_params=pltpu.CompilerParams(collective_id=0),
    )(x, dy)
